# baseline (speedup 1.0000x reference)
import functools

import jax
import jax.numpy as jnp
from jax import lax
from jax.experimental import pallas as pl
from jax.experimental.pallas import tpu as pltpu
from jax.experimental.pallas import tpu_sc as plsc


def _sc_gather(table, idx):
    V, D = table.shape
    (B,) = idx.shape
    info = plsc.get_sparse_core_info()
    NC, NS = info.num_cores, info.num_subcores
    NW = NC * NS
    b_per_w = B // NW
    mesh = plsc.VectorSubcoreMesh(core_axis_name="c", subcore_axis_name="s")

    @functools.partial(
        pl.kernel,
        mesh=mesh,
        compiler_params=pltpu.CompilerParams(use_tc_tiling_on_sc=False),
        out_type=jax.ShapeDtypeStruct((B, D), jnp.float32),
        scratch_types=[
            pltpu.VMEM((b_per_w,), jnp.int32),
            pltpu.VMEM((b_per_w, D), jnp.float32),
            pltpu.SemaphoreType.DMA,
        ],
    )
    def gather_kernel(table_hbm, idx_hbm, out_hbm, idx_v, rows_v, sem):
        wid = lax.axis_index("s") * NC + lax.axis_index("c")
        base = wid * b_per_w
        pltpu.sync_copy(idx_hbm.at[pl.ds(base, b_per_w)], idx_v)
        pltpu.async_copy(table_hbm.at[idx_v], rows_v, sem).wait()
        pltpu.sync_copy(rows_v, out_hbm.at[pl.ds(base, b_per_w)])

    return gather_kernel(table, idx)


def _tc_project(embeds, lin_w, lin_b, block_v):
    B, D = embeds.shape
    V = lin_w.shape[0]
    nv = pl.cdiv(V, block_v)
    lin_b2 = lin_b.reshape(1, V)

    def body(e_ref, w_ref, b_ref, o_ref):
        o_ref[...] = (
            lax.dot_general(
                e_ref[...],
                w_ref[...],
                dimension_numbers=(((1,), (1,)), ((), ())),
                preferred_element_type=jnp.float32,
            )
            + b_ref[...]
        )

    return pl.pallas_call(
        body,
        grid=(nv,),
        in_specs=[
            pl.BlockSpec((B, D), lambda j: (0, 0)),
            pl.BlockSpec((block_v, D), lambda j: (j, 0)),
            pl.BlockSpec((1, block_v), lambda j: (0, j)),
        ],
        out_specs=pl.BlockSpec((B, block_v), lambda j: (0, j)),
        out_shape=jax.ShapeDtypeStruct((B, V), jnp.float32),
    )(embeds, lin_w, lin_b2)


def kernel(inputs, emb_table, lin_w, lin_b):
    idx = inputs.astype(jnp.int32)
    embeds = _sc_gather(emb_table, idx)
    return _tc_project(embeds, lin_w, lin_b, block_v=2048)


# speedup vs baseline: 2.0441x; 2.0441x over previous
import functools

import jax
import jax.numpy as jnp
from jax import lax
from jax.experimental import pallas as pl
from jax.experimental.pallas import tpu as pltpu
from jax.experimental.pallas import tpu_sc as plsc


def _sc_gather(table, idx):
    V, D = table.shape
    (B,) = idx.shape
    info = plsc.get_sparse_core_info()
    NC, NS = info.num_cores, info.num_subcores
    NW = NC * NS
    b_per_w = B // NW
    mesh = plsc.VectorSubcoreMesh(core_axis_name="c", subcore_axis_name="s")

    @functools.partial(
        pl.kernel,
        mesh=mesh,
        compiler_params=pltpu.CompilerParams(use_tc_tiling_on_sc=False),
        out_type=jax.ShapeDtypeStruct((B, D), jnp.float32),
        scratch_types=[
            pltpu.VMEM((b_per_w,), jnp.int32),
            pltpu.VMEM((b_per_w, D), jnp.float32),
            pltpu.SemaphoreType.DMA,
        ],
    )
    def gather_kernel(table_hbm, idx_hbm, out_hbm, idx_v, rows_v, sem):
        wid = lax.axis_index("s") * NC + lax.axis_index("c")
        base = wid * b_per_w
        pltpu.sync_copy(idx_hbm.at[pl.ds(base, b_per_w)], idx_v)
        pltpu.async_copy(table_hbm.at[idx_v], rows_v, sem).wait()
        pltpu.sync_copy(rows_v, out_hbm.at[pl.ds(base, b_per_w)])

    return gather_kernel(table, idx)


def _tc_project(embeds, lin_w, lin_b, block_v):
    B, D = embeds.shape
    V = lin_w.shape[0]
    nv = pl.cdiv(V, block_v)
    eT = embeds.T
    lin_b2 = lin_b.reshape(V, 1)

    def body(w_ref, e_ref, b_ref, o_ref):
        o_ref[...] = (
            lax.dot_general(
                w_ref[...],
                e_ref[...],
                dimension_numbers=(((1,), (0,)), ((), ())),
                preferred_element_type=jnp.float32,
            )
            + b_ref[...]
        )

    outT = pl.pallas_call(
        body,
        grid=(nv,),
        in_specs=[
            pl.BlockSpec((block_v, D), lambda j: (j, 0)),
            pl.BlockSpec((D, B), lambda j: (0, 0)),
            pl.BlockSpec((block_v, 1), lambda j: (j, 0)),
        ],
        out_specs=pl.BlockSpec((block_v, B), lambda j: (j, 0)),
        out_shape=jax.ShapeDtypeStruct((V, B), jnp.float32),
    )(lin_w, eT, lin_b2)
    return outT.T


def kernel(inputs, emb_table, lin_w, lin_b):
    idx = inputs.astype(jnp.int32)
    embeds = _sc_gather(emb_table, idx)
    return _tc_project(embeds, lin_w, lin_b, block_v=2048)


# speedup vs baseline: 2.9575x; 1.4469x over previous
import functools

import jax
import jax.numpy as jnp
from jax import lax
from jax.experimental import pallas as pl
from jax.experimental.pallas import tpu as pltpu
from jax.experimental.pallas import tpu_sc as plsc


def _sc_gather(table, idx):
    V, D = table.shape
    (B,) = idx.shape
    info = plsc.get_sparse_core_info()
    NC, NS = info.num_cores, info.num_subcores
    NW = NC * NS
    b_per_w = B // NW
    mesh = plsc.VectorSubcoreMesh(core_axis_name="c", subcore_axis_name="s")

    @functools.partial(
        pl.kernel,
        mesh=mesh,
        compiler_params=pltpu.CompilerParams(use_tc_tiling_on_sc=False),
        out_type=jax.ShapeDtypeStruct((B, D), jnp.float32),
        scratch_types=[
            pltpu.VMEM((b_per_w,), jnp.int32),
            pltpu.VMEM((b_per_w, D), jnp.float32),
            pltpu.SemaphoreType.DMA,
        ],
    )
    def gather_kernel(table_hbm, idx_hbm, out_hbm, idx_v, rows_v, sem):
        wid = lax.axis_index("s") * NC + lax.axis_index("c")
        base = wid * b_per_w
        pltpu.sync_copy(idx_hbm.at[pl.ds(base, b_per_w)], idx_v)
        pltpu.async_copy(table_hbm.at[idx_v], rows_v, sem).wait()
        pltpu.sync_copy(rows_v, out_hbm.at[pl.ds(base, b_per_w)])

    return gather_kernel(table, idx)


def _tc_project(embeds, lin_w, lin_b, block_v):
    B, D = embeds.shape
    V = lin_w.shape[0]
    nv = pl.cdiv(V, block_v)
    waT = jnp.concatenate([lin_w.T, lin_b[None, :]], axis=0)
    eaT = jnp.concatenate([embeds.T, jnp.ones((1, B), jnp.float32)], axis=0)
    K = D + 1

    def body(w_ref, e_ref, o_ref):
        o_ref[...] = lax.dot_general(
            w_ref[...],
            e_ref[...],
            dimension_numbers=(((0,), (0,)), ((), ())),
            preferred_element_type=jnp.float32,
        )

    outT = pl.pallas_call(
        body,
        grid=(nv,),
        in_specs=[
            pl.BlockSpec((K, block_v), lambda j: (0, j)),
            pl.BlockSpec((K, B), lambda j: (0, 0)),
        ],
        out_specs=pl.BlockSpec((block_v, B), lambda j: (j, 0)),
        out_shape=jax.ShapeDtypeStruct((V, B), jnp.float32),
    )(waT, eaT)
    return outT.T


def kernel(inputs, emb_table, lin_w, lin_b):
    idx = inputs.astype(jnp.int32)
    embeds = _sc_gather(emb_table, idx)
    return _tc_project(embeds, lin_w, lin_b, block_v=2048)


# speedup vs baseline: 3.6253x; 1.2258x over previous
import functools

import jax
import jax.numpy as jnp
from jax import lax
from jax.experimental import pallas as pl
from jax.experimental.pallas import tpu as pltpu
from jax.experimental.pallas import tpu_sc as plsc


def _sc_gather_t(table_t_flat, idx, V, D, B):
    info = plsc.get_sparse_core_info()
    NC, NS, L = info.num_cores, info.num_subcores, info.num_lanes
    NW = NC * NS
    w_per_d = NW // D
    chunk = B // w_per_d
    n_idx_dma = chunk // 128
    mesh = plsc.VectorSubcoreMesh(core_axis_name="c", subcore_axis_name="s")

    @functools.partial(
        pl.kernel,
        mesh=mesh,
        compiler_params=pltpu.CompilerParams(use_tc_tiling_on_sc=False),
        out_type=jax.ShapeDtypeStruct((D, B), jnp.float32),
        scratch_types=[
            pltpu.VMEM((chunk,), jnp.int32),
            pltpu.VMEM((chunk,), jnp.int32),
            pltpu.VMEM((chunk,), jnp.float32),
            pltpu.SemaphoreType.DMA,
        ],
    )
    def gather_kernel(tflat_hbm, idx_hbm, out_hbm, idx_v, addr_v, dst_v, sem):
        wid = lax.axis_index("s") * NC + lax.axis_index("c")
        d = wid // w_per_d
        base = (wid % w_per_d) * chunk
        pltpu.sync_copy(idx_hbm.at[pl.ds(base, chunk)], idx_v)
        off = d * V
        for c in range(chunk // L):
            addr_v[pl.ds(c * L, L)] = idx_v[pl.ds(c * L, L)] + off
        copies = [
            pltpu.async_copy(
                tflat_hbm.at[addr_v.at[pl.ds(k * 128, 128)]],
                dst_v.at[pl.ds(k * 128, 128)],
                sem,
            )
            for k in range(n_idx_dma)
        ]
        for cp in copies:
            cp.wait()
        pltpu.sync_copy(dst_v, out_hbm.at[d, pl.ds(base, chunk)])

    return gather_kernel(table_t_flat, idx)


def _tc_project(eT, lin_w, lin_b, block_v):
    D, B = eT.shape
    V = lin_w.shape[0]
    nv = pl.cdiv(V, block_v)
    waT = jnp.concatenate([lin_w.T, lin_b[None, :]], axis=0)
    eaT = jnp.concatenate([eT, jnp.ones((1, B), jnp.float32)], axis=0)
    K = D + 1

    def body(w_ref, e_ref, o_ref):
        o_ref[...] = lax.dot_general(
            w_ref[...],
            e_ref[...],
            dimension_numbers=(((0,), (0,)), ((), ())),
            preferred_element_type=jnp.float32,
        )

    outT = pl.pallas_call(
        body,
        grid=(nv,),
        in_specs=[
            pl.BlockSpec((K, block_v), lambda j: (0, j)),
            pl.BlockSpec((K, B), lambda j: (0, 0)),
        ],
        out_specs=pl.BlockSpec((block_v, B), lambda j: (j, 0)),
        out_shape=jax.ShapeDtypeStruct((V, B), jnp.float32),
    )(waT, eaT)
    return outT.T


def kernel(inputs, emb_table, lin_w, lin_b):
    idx = inputs.astype(jnp.int32)
    V, D = emb_table.shape
    (B,) = idx.shape
    table_t_flat = emb_table.T.reshape(-1)
    eT = _sc_gather_t(table_t_flat, idx, V, D, B)
    return _tc_project(eT, lin_w, lin_b, block_v=2048)


# speedup vs baseline: 3.7702x; 1.0400x over previous
import functools

import jax
import jax.numpy as jnp
from jax import lax
from jax.experimental import pallas as pl
from jax.experimental.pallas import tpu as pltpu
from jax.experimental.pallas import tpu_sc as plsc


def _sc_gather_t(table_t_flat, idx, V, D, B):
    info = plsc.get_sparse_core_info()
    NC, NS, L = info.num_cores, info.num_subcores, info.num_lanes
    NW = NC * NS
    w_per_d = NW // D
    chunk = B // w_per_d
    n_idx_dma = chunk // 128
    mesh = plsc.VectorSubcoreMesh(core_axis_name="c", subcore_axis_name="s")

    @functools.partial(
        pl.kernel,
        mesh=mesh,
        compiler_params=pltpu.CompilerParams(use_tc_tiling_on_sc=False),
        out_type=jax.ShapeDtypeStruct((D, B), jnp.float32),
        scratch_types=[
            pltpu.VMEM((chunk,), jnp.int32),
            pltpu.VMEM((chunk,), jnp.int32),
            pltpu.VMEM((chunk,), jnp.float32),
            pltpu.SemaphoreType.DMA,
        ],
    )
    def gather_kernel(tflat_hbm, idx_hbm, out_hbm, idx_v, addr_v, dst_v, sem):
        wid = lax.axis_index("s") * NC + lax.axis_index("c")
        d = wid // w_per_d
        base = (wid % w_per_d) * chunk
        pltpu.sync_copy(idx_hbm.at[pl.ds(base, chunk)], idx_v)
        off = d * V
        for c in range(chunk // L):
            addr_v[pl.ds(c * L, L)] = idx_v[pl.ds(c * L, L)] + off
        copies = [
            pltpu.async_copy(
                tflat_hbm.at[addr_v.at[pl.ds(k * 128, 128)]],
                dst_v.at[pl.ds(k * 128, 128)],
                sem,
            )
            for k in range(n_idx_dma)
        ]
        for cp in copies:
            cp.wait()
        pltpu.sync_copy(dst_v, out_hbm.at[d, pl.ds(base, chunk)])

    return gather_kernel(table_t_flat, idx)


def _tc_project(eT, lin_w, lin_b, block_v):
    D, B = eT.shape
    V = lin_w.shape[0]
    nv = pl.cdiv(V, block_v)
    wT = lin_w.T
    lin_b2 = lin_b.reshape(1, V)

    def body(w_ref, b_ref, e_ref, o_ref):
        wa = jnp.concatenate([w_ref[...], b_ref[...]], axis=0)
        ea = jnp.concatenate([e_ref[...], jnp.ones((1, B), jnp.float32)], axis=0)
        o_ref[...] = lax.dot_general(
            wa,
            ea,
            dimension_numbers=(((0,), (0,)), ((), ())),
            preferred_element_type=jnp.float32,
        )

    outT = pl.pallas_call(
        body,
        grid=(nv,),
        in_specs=[
            pl.BlockSpec((D, block_v), lambda j: (0, j)),
            pl.BlockSpec((1, block_v), lambda j: (0, j)),
            pl.BlockSpec((D, B), lambda j: (0, 0)),
        ],
        out_specs=pl.BlockSpec((block_v, B), lambda j: (j, 0)),
        out_shape=jax.ShapeDtypeStruct((V, B), jnp.float32),
    )(wT, lin_b2, eT)
    return outT.T


def kernel(inputs, emb_table, lin_w, lin_b):
    idx = inputs.astype(jnp.int32)
    V, D = emb_table.shape
    (B,) = idx.shape
    table_t_flat = emb_table.T.reshape(-1)
    eT = _sc_gather_t(table_t_flat, idx, V, D, B)
    return _tc_project(eT, lin_w, lin_b, block_v=2048)
